# baseline (speedup 1.0000x reference)
import functools

import jax
import jax.numpy as jnp
from jax import lax
from jax.experimental import pallas as pl
from jax.experimental.pallas import tpu as pltpu
from jax.experimental.pallas import tpu_sc as plsc

B, T, H = 1024, 200, 64
V = 100000
BT = B * T
NW = 32
CHUNK = 128

RK = BT * 3
RO = BT * 2
RS = B * 3
PW_K, PW_O, PW_S = RK // NW, RO // NW, RS // NW


def _cont_body(kc, oc, tg, ke, kb, oe, ob, te, tb, outk, outo, outt):
    for j in range(4):
        outk[:, 3 + j, :] = kc[:, j:j + 1] * ke[j:j + 1, :] + kb[j:j + 1, :]
    for j in range(6):
        outo[:, 2 + j, :] = oc[:, j:j + 1] * oe[j:j + 1, :] + ob[j:j + 1, :]
    outt[:, 0, :] = tg[:, 0:1] * te[0:1, :] + tb[0:1, :]


def _s_cont_body(sc, se, sb, outs):
    for j in range(4):
        outs[:, 3 + j, :] = sc[:, j:j + 1] * se[j:j + 1, :] + sb[j:j + 1, :]


def _tc_cont(k_cont, o_cont, target, ke, kb, oe, ob, te, tb, blk=2048):
    grid = (BT // blk,)
    full = lambda s: pl.BlockSpec(s, lambda i: (0, 0))
    row = lambda n: pl.BlockSpec((blk, n), lambda i: (i, 0))
    out3 = lambda f: pl.BlockSpec((blk, f, H), lambda i: (i, 0, 0))
    return pl.pallas_call(
        _cont_body,
        grid=grid,
        in_specs=[row(4), row(6), row(1),
                  full((4, H)), full((4, H)), full((6, H)), full((6, H)),
                  full((1, H)), full((1, H))],
        out_specs=[out3(7), out3(8), out3(1)],
        out_shape=[jax.ShapeDtypeStruct((BT, 7, H), jnp.float32),
                   jax.ShapeDtypeStruct((BT, 8, H), jnp.float32),
                   jax.ShapeDtypeStruct((BT, 1, H), jnp.float32)],
    )(k_cont, o_cont, target, ke, kb, oe, ob, te, tb)


def _tc_s_cont(s_cont, se, sb):
    full = lambda s: pl.BlockSpec(s, lambda: (0, 0))
    return pl.pallas_call(
        _s_cont_body,
        in_specs=[pl.BlockSpec((B, 4), lambda: (0, 0)),
                  full((4, H)), full((4, H))],
        out_specs=pl.BlockSpec((B, 7, H), lambda: (0, 0, 0)),
        out_shape=jax.ShapeDtypeStruct((B, 7, H), jnp.float32),
    )(s_cont, se, sb)


def _sc_body(gk, dk, go, do_, gs, ds, tk, to, ts, outk, outo, outs,
             idx_v, dst_v, row_v, idx_s, dst_s, row_s, sem):
    w = lax.axis_index("s") * 2 + lax.axis_index("c")

    def run(gidx, didx, tab, out, per_w, n_chunks):
        wbase = w * per_w

        def step(i, carry):
            base = wbase + i * CHUNK
            pltpu.sync_copy(gidx.at[pl.ds(base, CHUNK)], idx_v)
            pltpu.sync_copy(didx.at[pl.ds(base, CHUNK)], dst_v)
            pltpu.async_copy(tab.at[idx_v], row_v, sem).wait()
            pltpu.async_copy(row_v, out.at[dst_v], sem).wait()
            return carry

        lax.fori_loop(0, n_chunks, step, 0)

    run(gk, dk, tk, outk, PW_K, PW_K // CHUNK)
    run(go, do_, to, outo, PW_O, PW_O // CHUNK)
    base = w * PW_S
    pltpu.sync_copy(gs.at[pl.ds(base, PW_S)], idx_s)
    pltpu.sync_copy(ds.at[pl.ds(base, PW_S)], dst_s)
    pltpu.async_copy(ts.at[idx_s], row_s, sem).wait()
    pltpu.async_copy(row_s, outs.at[dst_s], sem).wait()


_sc_fill = functools.partial(
    pl.kernel,
    _sc_body,
    out_type=(),
    mesh=plsc.VectorSubcoreMesh(core_axis_name="c", subcore_axis_name="s"),
    compiler_params=pltpu.CompilerParams(use_tc_tiling_on_sc=False),
    scratch_types=[
        pltpu.VMEM((CHUNK,), jnp.int32),
        pltpu.VMEM((CHUNK,), jnp.int32),
        pltpu.VMEM((CHUNK, H), jnp.float32),
        pltpu.VMEM((PW_S,), jnp.int32),
        pltpu.VMEM((PW_S,), jnp.int32),
        pltpu.VMEM((PW_S, H), jnp.float32),
        pltpu.SemaphoreType.DMA,
    ],
)()


def kernel(s_cat, s_cont, k_cat, k_cont, o_cat, o_cont, target,
           s_cat_tables, k_cat_tables, o_cat_tables,
           s_cont_emb, s_cont_bias, k_cont_emb, k_cont_bias,
           o_cont_emb, o_cont_bias, tgt_emb, tgt_bias):
    k_full, o_full, t_full = _tc_cont(
        k_cont.reshape(BT, 4), o_cont.reshape(BT, 6), target.reshape(BT, 1),
        k_cont_emb, k_cont_bias, o_cont_emb, o_cont_bias, tgt_emb, tgt_bias)
    s_full = _tc_s_cont(s_cont[:, 0, :], s_cont_emb, s_cont_bias)

    i32 = jnp.int32
    f3 = (jnp.arange(3, dtype=i32) * V)[None, :]
    f2 = (jnp.arange(2, dtype=i32) * V)[None, :]
    gidx_k = (k_cat.reshape(BT, 3) + f3).reshape(RK)
    gidx_o = (o_cat.reshape(BT, 2) + f2).reshape(RO)
    gidx_s = (s_cat[:, 0, :] + f3).reshape(RS)
    rows_bt = jnp.arange(BT, dtype=i32)[:, None]
    didx_k = (rows_bt * 7 + jnp.arange(3, dtype=i32)[None, :]).reshape(RK)
    didx_o = (rows_bt * 8 + jnp.arange(2, dtype=i32)[None, :]).reshape(RO)
    didx_s = (jnp.arange(B, dtype=i32)[:, None] * 7
              + jnp.arange(3, dtype=i32)[None, :]).reshape(RS)

    outk = jax.new_ref(k_full.reshape(BT * 7, H))
    outo = jax.new_ref(o_full.reshape(BT * 8, H))
    outs = jax.new_ref(s_full.reshape(B * 7, H))
    _sc_fill(gidx_k, didx_k, gidx_o, didx_o, gidx_s, didx_s,
             k_cat_tables.reshape(3 * V, H),
             o_cat_tables.reshape(2 * V, H),
             s_cat_tables.reshape(3 * V, H),
             outk, outo, outs)

    return (outs[...].reshape(B, 7, H),
            outk[...].reshape(B, T, 7, H),
            outo[...].reshape(B, T, 8, H),
            t_full.reshape(B, T, 1, H))


# speedup vs baseline: 1.8829x; 1.8829x over previous
import functools

import jax
import jax.numpy as jnp
from jax import lax
from jax.experimental import pallas as pl
from jax.experimental.pallas import tpu as pltpu
from jax.experimental.pallas import tpu_sc as plsc

B, T, H = 1024, 200, 64
V = 100000
BT = B * T
NW = 32
CHUNK = 128

RK = BT * 3
RO = BT * 2
RS = B * 3
PW_K, PW_O, PW_S = RK // NW, RO // NW, RS // NW
NCH_K, NCH_O = PW_K // CHUNK, PW_O // CHUNK


def _sc_body(gk, dk, go, do_, gs, ds, tk, to, ts, outk, outo, outs,
             idx_all, dst_all, row0, row1, idx_s, dst_s, row_s,
             g0, g1, s0, s1, sem):
    w = lax.axis_index("s") * 2 + lax.axis_index("c")
    rows = (row0, row1)
    gsem = (g0, g1)
    ssem = (s0, s1)

    def run(gidx, didx, tab, out, n):
        pltpu.sync_copy(gidx.at[w], idx_all.at[pl.ds(0, n)])
        pltpu.sync_copy(didx.at[w], dst_all.at[pl.ds(0, n)])

        def gather(i, p):
            return pltpu.make_async_copy(tab.at[idx_all.at[i]], rows[p],
                                         gsem[p])

        def scatter(i, p):
            return pltpu.make_async_copy(rows[p], out.at[dst_all.at[i]],
                                         ssem[p])

        gather(0, 0).start()

        def step(j, carry):
            i0 = 2 * j
            i1 = i0 + 1
            gather(i0, 0).wait()

            @pl.when(j >= 1)
            def _():
                scatter(i0 - 1, 1).wait()

            gather(i1, 1).start()
            scatter(i0, 0).start()
            gather(i1, 1).wait()
            scatter(i0, 0).wait()

            @pl.when(i1 + 1 < n)
            def _():
                gather(i1 + 1, 0).start()

            scatter(i1, 1).start()
            return carry

        lax.fori_loop(0, n // 2, step, 0)
        scatter(n - 1, 1).wait()

    run(gk, dk, tk, outk, NCH_K)
    run(go, do_, to, outo, NCH_O)

    base = w * PW_S
    pltpu.sync_copy(gs.at[pl.ds(base, PW_S)], idx_s)
    pltpu.sync_copy(ds.at[pl.ds(base, PW_S)], dst_s)
    pltpu.async_copy(ts.at[idx_s], row_s, sem).wait()
    pltpu.async_copy(row_s, outs.at[dst_s], sem).wait()


_sc_fill = pl.kernel(
    _sc_body,
    out_type=(jax.ShapeDtypeStruct((2 * RK, H), jnp.float32),
              jax.ShapeDtypeStruct((2 * RO, H), jnp.float32),
              jax.ShapeDtypeStruct((2 * RS, H), jnp.float32)),
    mesh=plsc.VectorSubcoreMesh(core_axis_name="c", subcore_axis_name="s"),
    compiler_params=pltpu.CompilerParams(use_tc_tiling_on_sc=False),
    scratch_types=[
        pltpu.VMEM((NCH_K, CHUNK), jnp.int32),
        pltpu.VMEM((NCH_K, CHUNK), jnp.int32),
        pltpu.VMEM((CHUNK, H), jnp.float32),
        pltpu.VMEM((CHUNK, H), jnp.float32),
        pltpu.VMEM((PW_S,), jnp.int32),
        pltpu.VMEM((PW_S,), jnp.int32),
        pltpu.VMEM((PW_S, H), jnp.float32),
        pltpu.SemaphoreType.DMA,
        pltpu.SemaphoreType.DMA,
        pltpu.SemaphoreType.DMA,
        pltpu.SemaphoreType.DMA,
        pltpu.SemaphoreType.DMA,
    ],
)


def _asm_body(ck, co, kc, oc, tg, ke, kb, oe, ob, te, tb, outk, outo, outt):
    for f in range(3):
        outk[:, f, :] = ck[f, :, :H]
    for j in range(4):
        outk[:, 3 + j, :] = kc[:, j:j + 1] * ke[j:j + 1, :] + kb[j:j + 1, :]
    for f in range(2):
        outo[:, f, :] = co[f, :, :H]
    for j in range(6):
        outo[:, 2 + j, :] = oc[:, j:j + 1] * oe[j:j + 1, :] + ob[j:j + 1, :]
    outt[:, 0, :] = tg[:, 0:1] * te[0:1, :] + tb[0:1, :]


def _tc_assemble(catk, cato, k_cont, o_cont, target,
                 ke, kb, oe, ob, te, tb, blk=2048):
    grid = (BT // blk,)
    full = lambda s: pl.BlockSpec(s, lambda i: (0, 0))
    row = lambda n: pl.BlockSpec((blk, n), lambda i: (i, 0))
    b3 = lambda f: pl.BlockSpec((f, blk, 2 * H), lambda i: (0, i, 0))
    out3 = lambda f: pl.BlockSpec((blk, f, H), lambda i: (i, 0, 0))
    return pl.pallas_call(
        _asm_body,
        grid=grid,
        in_specs=[b3(3), b3(2), row(4), row(6), row(1),
                  full((4, H)), full((4, H)), full((6, H)), full((6, H)),
                  full((1, H)), full((1, H))],
        out_specs=[out3(7), out3(8), out3(1)],
        out_shape=[jax.ShapeDtypeStruct((BT, 7, H), jnp.float32),
                   jax.ShapeDtypeStruct((BT, 8, H), jnp.float32),
                   jax.ShapeDtypeStruct((BT, 1, H), jnp.float32)],
    )(catk, cato, k_cont, o_cont, target, ke, kb, oe, ob, te, tb)


def _s_asm_body(cs, sc, se, sb, outs):
    for f in range(3):
        outs[:, f, :] = cs[f, :, :H]
    for j in range(4):
        outs[:, 3 + j, :] = sc[:, j:j + 1] * se[j:j + 1, :] + sb[j:j + 1, :]


def _tc_s_assemble(cats, s_cont, se, sb):
    f2 = lambda s: pl.BlockSpec(s, lambda: (0, 0))
    return pl.pallas_call(
        _s_asm_body,
        in_specs=[pl.BlockSpec((3, B, 2 * H), lambda: (0, 0, 0)),
                  f2((B, 4)), f2((4, H)), f2((4, H))],
        out_specs=pl.BlockSpec((B, 7, H), lambda: (0, 0, 0)),
        out_shape=jax.ShapeDtypeStruct((B, 7, H), jnp.float32),
    )(cats, s_cont, se, sb)


def kernel(s_cat, s_cont, k_cat, k_cont, o_cat, o_cont, target,
           s_cat_tables, k_cat_tables, o_cat_tables,
           s_cont_emb, s_cont_bias, k_cont_emb, k_cont_bias,
           o_cont_emb, o_cont_bias, tgt_emb, tgt_bias):
    i32 = jnp.int32
    f3 = (jnp.arange(3, dtype=i32) * V)[None, :]
    f2 = (jnp.arange(2, dtype=i32) * V)[None, :]
    gidx_k = (k_cat.reshape(BT, 3) + f3).reshape(NW, NCH_K, CHUNK)
    gidx_o = (o_cat.reshape(BT, 2) + f2).reshape(NW, NCH_O, CHUNK)
    gidx_s = (s_cat[:, 0, :] + f3).reshape(RS)
    rows_bt = jnp.arange(BT, dtype=i32)[:, None]
    didx_k = (2 * (jnp.arange(3, dtype=i32)[None, :] * BT + rows_bt)
              ).reshape(NW, NCH_K, CHUNK)
    didx_o = (2 * (jnp.arange(2, dtype=i32)[None, :] * BT + rows_bt)
              ).reshape(NW, NCH_O, CHUNK)
    didx_s = (2 * (jnp.arange(3, dtype=i32)[None, :] * B
                   + jnp.arange(B, dtype=i32)[:, None])).reshape(RS)

    catk, cato, cats = _sc_fill(
        gidx_k, didx_k, gidx_o, didx_o, gidx_s, didx_s,
        k_cat_tables.reshape(3 * V, H),
        o_cat_tables.reshape(2 * V, H),
        s_cat_tables.reshape(3 * V, H))

    catk = catk.reshape(3, BT, 2 * H)
    cato = cato.reshape(2, BT, 2 * H)
    cats = cats.reshape(3, B, 2 * H)

    k_full, o_full, t_full = _tc_assemble(
        catk, cato, k_cont.reshape(BT, 4), o_cont.reshape(BT, 6),
        target.reshape(BT, 1), k_cont_emb, k_cont_bias,
        o_cont_emb, o_cont_bias, tgt_emb, tgt_bias)
    s_full = _tc_s_assemble(cats, s_cont[:, 0, :], s_cont_emb, s_cont_bias)

    return (s_full,
            k_full.reshape(B, T, 7, H),
            o_full.reshape(B, T, 8, H),
            t_full.reshape(B, T, 1, H))
